# Initial kernel scaffold; baseline (speedup 1.0000x reference)
import jax
import jax.numpy as jnp
from jax.experimental import pallas as pl


def kernel(x, edge_index, edge_attr, batch, W1e, b1e, W1, b1, g1, bt1, W2e, b2e, W2, b2, g2, bt2, W3e, b3e, W3, b3, g3, bt3, Wl, bl):
    raise NotImplementedError("write your pallas kernel here")


# speedup vs baseline: 2.9015x; 2.9015x over previous
import functools

import jax
import jax.numpy as jnp
from jax import lax
from jax.experimental import pallas as pl
from jax.experimental.pallas import tpu as pltpu
from jax.experimental.pallas import tpu_sc as plsc

_N = 10000
_D = 128
_E = 320000
_G = 64
_LANES = 16
_CHUNK = 128
_NCH = _E // _CHUNK
_NW = 32
_STRIPE = 624
_TAIL = _N - 16 * _STRIPE


def _edge_mlp_body(ea_ref, w1_ref, b1_ref, w2_ref, b2_ref, w3_ref, b3_ref,
                   o1_ref, o2_ref, o3_ref):
    ea = ea_ref[...]
    o1_ref[...] = jnp.dot(ea, w1_ref[...],
                          preferred_element_type=jnp.float32) + b1_ref[...]
    o2_ref[...] = jnp.dot(ea, w2_ref[...],
                          preferred_element_type=jnp.float32) + b2_ref[...]
    o3_ref[...] = jnp.dot(ea, w3_ref[...],
                          preferred_element_type=jnp.float32) + b3_ref[...]


def _edge_mlp(edge_attr, w1, b1, w2, b2, w3, b3):
    blk = 2000
    grid = _E // blk
    ed = edge_attr.shape[1]
    out_spec = pl.BlockSpec((blk, _D), lambda i: (i, 0))
    w_spec = pl.BlockSpec((ed, _D), lambda i: (0, 0))
    b_spec = pl.BlockSpec((1, _D), lambda i: (0, 0))
    return pl.pallas_call(
        _edge_mlp_body,
        grid=(grid,),
        in_specs=[pl.BlockSpec((blk, ed), lambda i: (i, 0)),
                  w_spec, b_spec, w_spec, b_spec, w_spec, b_spec],
        out_specs=[out_spec, out_spec, out_spec],
        out_shape=[jax.ShapeDtypeStruct((_E, _D), jnp.float32)] * 3,
    )(edge_attr, w1, b1.reshape(1, _D), w2, b2.reshape(1, _D),
      w3, b3.reshape(1, _D))


def _sc_edge_body(x_hbm, e_hbm, src_hbm, dst_hbm, out_hbm,
                  src_v, dst_v, xbuf, ebuf, acc, sem):
    cid = lax.axis_index("c")
    sid = lax.axis_index("s")
    wid = sid * 2 + cid

    def zero_row(r, _):
        for j in range(_D // _LANES):
            ebuf[r, pl.ds(j * _LANES, _LANES)] = jnp.zeros(
                (_LANES,), jnp.float32)
        return 0

    lax.fori_loop(0, _CHUNK, zero_row, 0)
    for k in range(4):
        pltpu.sync_copy(ebuf, acc.at[pl.ds(sid * _STRIPE + k * _CHUNK,
                                           _CHUNK)])
    pltpu.sync_copy(ebuf.at[pl.ds(0, _STRIPE - 4 * _CHUNK)],
                    acc.at[pl.ds(sid * _STRIPE + 4 * _CHUNK,
                                 _STRIPE - 4 * _CHUNK)])

    @pl.when(sid == 15)
    def _zero_tail():
        pltpu.sync_copy(ebuf.at[pl.ds(0, _TAIL)],
                        acc.at[pl.ds(16 * _STRIPE, _TAIL)])

    plsc.subcore_barrier()

    nch = 78 + jnp.where(wid < _NCH - 78 * _NW, 1, 0)

    def chunk_body(i, _):
        base = (i * _NW + wid) * _CHUNK
        pltpu.sync_copy(src_hbm.at[pl.ds(base, _CHUNK)], src_v)
        pltpu.sync_copy(dst_hbm.at[pl.ds(base, _CHUNK)], dst_v)
        pltpu.async_copy(x_hbm.at[src_v], xbuf, sem).wait()
        pltpu.sync_copy(e_hbm.at[pl.ds(base, _CHUNK)], ebuf)

        def row_body(r, _):
            for j in range(_D // _LANES):
                s = pl.ds(j * _LANES, _LANES)
                ebuf[r, s] = jnp.maximum(ebuf[r, s] + xbuf[r, s], 0.0)
            return 0

        lax.fori_loop(0, _CHUNK, row_body, 0)
        pltpu.sync_copy(ebuf, acc.at[dst_v], add=True)
        return 0

    lax.fori_loop(0, nch, chunk_body, 0)
    plsc.subcore_barrier()

    row0 = sid * _STRIPE
    pltpu.sync_copy(acc.at[pl.ds(row0, _STRIPE)],
                    out_hbm.at[pl.ds(cid * _N + row0, _STRIPE)])

    @pl.when(sid == 15)
    def _write_tail():
        pltpu.sync_copy(acc.at[pl.ds(16 * _STRIPE, _TAIL)],
                        out_hbm.at[pl.ds(cid * _N + 16 * _STRIPE, _TAIL)])


def _sc_edge(x, e_l, src, dst):
    mesh = plsc.VectorSubcoreMesh(core_axis_name="c", subcore_axis_name="s")
    f = functools.partial(
        pl.kernel,
        out_type=jax.ShapeDtypeStruct((2 * _N, _D), jnp.float32),
        mesh=mesh,
        scratch_types=[
            pltpu.VMEM((_CHUNK,), jnp.int32),
            pltpu.VMEM((_CHUNK,), jnp.int32),
            pltpu.VMEM((_CHUNK, _D), jnp.float32),
            pltpu.VMEM((_CHUNK, _D), jnp.float32),
            pltpu.VMEM_SHARED((_N, _D), jnp.float32),
            pltpu.SemaphoreType.DMA,
        ],
    )(_sc_edge_body)
    return f(x, e_l, src, dst)


def _node_body(x_ref, agg_ref, w_ref, b_ref, g_ref, bt_ref, o_ref):
    xa = x_ref[...] + agg_ref[0:_N, :] + agg_ref[_N:2 * _N, :]
    h = jnp.dot(xa, w_ref[...], preferred_element_type=jnp.float32) + b_ref[...]
    mu = jnp.mean(h, axis=0, keepdims=True)
    var = jnp.mean(h * h, axis=0, keepdims=True) - mu * mu
    hn = (h - mu) * lax.rsqrt(var + 1e-5) * g_ref[...] + bt_ref[...]
    o_ref[...] = jnp.where(hn >= 0, hn, 0.01 * hn)


def _node(x, agg2, w, b, g, bt):
    return pl.pallas_call(
        _node_body,
        out_shape=jax.ShapeDtypeStruct((_N, _D), jnp.float32),
    )(x, agg2, w, b.reshape(1, _D), g.reshape(1, _D), bt.reshape(1, _D))


def _final_body(h1_ref, h2_ref, h3_ref, brow_ref, bcol_ref, wl_ref, bl_ref,
                o_ref):
    w1 = wl_ref[0:_D, :]
    w2 = wl_ref[_D:2 * _D, :]
    w3 = wl_ref[2 * _D:3 * _D, :]
    w4 = wl_ref[3 * _D:4 * _D, :]
    h3 = h3_ref[...]
    t = (jnp.dot(h1_ref[...], w1, preferred_element_type=jnp.float32)
         + jnp.dot(h2_ref[...], w2, preferred_element_type=jnp.float32)
         + jnp.dot(h3, w3, preferred_element_type=jnp.float32))
    oh = (lax.broadcasted_iota(jnp.int32, (_G, _N), 0)
          == brow_ref[...]).astype(jnp.float32)
    pool = jnp.dot(oh, h3, preferred_element_type=jnp.float32)
    s = jnp.dot(pool, w4, preferred_element_type=jnp.float32)
    oht = (lax.broadcasted_iota(jnp.int32, (_N, _G), 1)
           == bcol_ref[...]).astype(jnp.float32)
    pooled = jnp.dot(oht, s, preferred_element_type=jnp.float32)
    z = t + pooled + bl_ref[...]
    o_ref[...] = 1.0 / (1.0 + jnp.exp(-z))


def _final(h1, h2, h3, batch, wl, bl):
    return pl.pallas_call(
        _final_body,
        out_shape=jax.ShapeDtypeStruct((_N, 1), jnp.float32),
    )(h1, h2, h3, batch.reshape(1, _N), batch.reshape(_N, 1), wl,
      bl.reshape(1, 1))


def kernel(x, edge_index, edge_attr, batch,
           W1e, b1e, W1, b1, g1, bt1,
           W2e, b2e, W2, b2, g2, bt2,
           W3e, b3e, W3, b3, g3, bt3,
           Wl, bl):
    src = edge_index[0]
    dst = edge_index[1]
    e1, e2, e3 = _edge_mlp(edge_attr, W1e, b1e, W2e, b2e, W3e, b3e)

    h = x
    hs = []
    for e_l, w, b, g, bt in ((e1, W1, b1, g1, bt1),
                             (e2, W2, b2, g2, bt2),
                             (e3, W3, b3, g3, bt3)):
        agg2 = _sc_edge(h, e_l, src, dst)
        h = _node(h, agg2, w, b, g, bt)
        hs.append(h)

    return _final(hs[0], hs[1], hs[2], batch, Wl, bl)


# speedup vs baseline: 5.0552x; 1.7423x over previous
import functools

import jax
import jax.numpy as jnp
from jax import lax
from jax.experimental import pallas as pl
from jax.experimental.pallas import tpu as pltpu
from jax.experimental.pallas import tpu_sc as plsc

_N = 10000
_D = 128
_E = 320000
_G = 64
_LANES = 16
_CHUNK = 64
_NCH = _E // _CHUNK
_NSLOT = 3
_NW = 32
_STRIPE = 624
_TAIL = _N - 16 * _STRIPE


def _edge_mlp_body(ea_ref, w1_ref, b1_ref, w2_ref, b2_ref, w3_ref, b3_ref,
                   o1_ref, o2_ref, o3_ref):
    ea = ea_ref[...]
    o1_ref[...] = jnp.dot(ea, w1_ref[...],
                          preferred_element_type=jnp.float32) + b1_ref[...]
    o2_ref[...] = jnp.dot(ea, w2_ref[...],
                          preferred_element_type=jnp.float32) + b2_ref[...]
    o3_ref[...] = jnp.dot(ea, w3_ref[...],
                          preferred_element_type=jnp.float32) + b3_ref[...]


def _edge_mlp(edge_attr, w1, b1, w2, b2, w3, b3):
    blk = 2000
    grid = _E // blk
    ed = edge_attr.shape[1]
    out_spec = pl.BlockSpec((blk, _D), lambda i: (i, 0))
    w_spec = pl.BlockSpec((ed, _D), lambda i: (0, 0))
    b_spec = pl.BlockSpec((1, _D), lambda i: (0, 0))
    return pl.pallas_call(
        _edge_mlp_body,
        grid=(grid,),
        in_specs=[pl.BlockSpec((blk, ed), lambda i: (i, 0)),
                  w_spec, b_spec, w_spec, b_spec, w_spec, b_spec],
        out_specs=[out_spec, out_spec, out_spec],
        out_shape=[jax.ShapeDtypeStruct((_E, _D), jnp.float32)] * 3,
    )(edge_attr, w1, b1.reshape(1, _D), w2, b2.reshape(1, _D),
      w3, b3.reshape(1, _D))


def _sc_edge_body(x_hbm, e_hbm, src_hbm, dst_hbm, out_hbm, *sc):
    src_v = sc[0:3]
    dst_v = sc[3:6]
    xbuf = sc[6:9]
    ebuf = sc[9:12]
    acc = sc[12]
    sem_is = sc[13:16]
    sem_id = sc[16:19]
    sem_g = sc[19:22]
    sem_e = sc[22:25]
    sem_sc = sc[25:28]

    cid = lax.axis_index("c")
    sid = lax.axis_index("s")
    wid = sid * 2 + cid

    def zero_row(r, _):
        for j in range(_D // _LANES):
            ebuf[0][r, pl.ds(j * _LANES, _LANES)] = jnp.zeros(
                (_LANES,), jnp.float32)
        return 0

    lax.fori_loop(0, _CHUNK, zero_row, 0)
    for k in range(9):
        pltpu.sync_copy(ebuf[0], acc.at[pl.ds(sid * _STRIPE + k * _CHUNK,
                                              _CHUNK)])
    pltpu.sync_copy(ebuf[0].at[pl.ds(0, _STRIPE - 9 * _CHUNK)],
                    acc.at[pl.ds(sid * _STRIPE + 9 * _CHUNK,
                                 _STRIPE - 9 * _CHUNK)])

    @pl.when(sid == 15)
    def _zero_tail():
        pltpu.sync_copy(ebuf[0].at[pl.ds(0, _TAIL)],
                        acc.at[pl.ds(16 * _STRIPE, _TAIL)])

    plsc.subcore_barrier()

    nch = 156 + jnp.where(wid < _NCH - 156 * _NW, 1, 0)

    def issue_loads(k, slot):
        base = (k * _NW + wid) * _CHUNK
        pltpu.async_copy(src_hbm.at[pl.ds(base, _CHUNK)], src_v[slot],
                         sem_is[slot])
        pltpu.async_copy(dst_hbm.at[pl.ds(base, _CHUNK)], dst_v[slot],
                         sem_id[slot])
        pltpu.async_copy(e_hbm.at[pl.ds(base, _CHUNK)], ebuf[slot],
                         sem_e[slot])

    def issue_gather(slot):
        pltpu.make_async_copy(src_hbm.at[pl.ds(0, _CHUNK)], src_v[slot],
                              sem_is[slot]).wait()
        pltpu.async_copy(x_hbm.at[src_v[slot]], xbuf[slot], sem_g[slot])

    issue_loads(jnp.int32(0), 0)
    issue_loads(jnp.int32(1), 1)
    issue_gather(0)

    def tri_body(i3, _):
        for u in range(_NSLOT):
            k = i3 * _NSLOT + u
            s1 = (u + 1) % _NSLOT
            sj = (u + 2) % _NSLOT
            j = k + 2

            @pl.when((k >= 1) & (j < nch))
            def _refill_wait():
                pltpu.make_async_copy(ebuf[sj], acc.at[pl.ds(0, _CHUNK)],
                                      sem_sc[sj]).wait()
                issue_loads(j, sj)

            @pl.when((k == 0) & (j < nch))
            def _refill_fresh():
                issue_loads(j, sj)

            @pl.when(k + 1 < nch)
            def _gather_next():
                issue_gather(s1)

            @pl.when(k < nch)
            def _process():
                pltpu.make_async_copy(x_hbm.at[pl.ds(0, _CHUNK)], xbuf[u],
                                      sem_g[u]).wait()
                pltpu.make_async_copy(e_hbm.at[pl.ds(0, _CHUNK)], ebuf[u],
                                      sem_e[u]).wait()

                def row_body(r, _):
                    for jj in range(_D // _LANES):
                        s = pl.ds(jj * _LANES, _LANES)
                        ebuf[u][r, s] = jnp.maximum(
                            ebuf[u][r, s] + xbuf[u][r, s], 0.0)
                    return 0

                lax.fori_loop(0, _CHUNK, row_body, 0)
                pltpu.make_async_copy(dst_hbm.at[pl.ds(0, _CHUNK)], dst_v[u],
                                      sem_id[u]).wait()
                pltpu.async_copy(ebuf[u], acc.at[dst_v[u]], sem_sc[u],
                                 add=True)
        return 0

    lax.fori_loop(0, (_NCH // _NW + _NSLOT) // _NSLOT, tri_body, 0)

    for u in range(_NSLOT):
        pltpu.make_async_copy(ebuf[u], acc.at[pl.ds(0, _CHUNK)],
                              sem_sc[u]).wait()

    plsc.subcore_barrier()

    row0 = sid * _STRIPE
    pltpu.sync_copy(acc.at[pl.ds(row0, _STRIPE)],
                    out_hbm.at[pl.ds(cid * _N + row0, _STRIPE)])

    @pl.when(sid == 15)
    def _write_tail():
        pltpu.sync_copy(acc.at[pl.ds(16 * _STRIPE, _TAIL)],
                        out_hbm.at[pl.ds(cid * _N + 16 * _STRIPE, _TAIL)])


def _sc_edge(x, e_l, src, dst):
    mesh = plsc.VectorSubcoreMesh(core_axis_name="c", subcore_axis_name="s")
    f = functools.partial(
        pl.kernel,
        out_type=jax.ShapeDtypeStruct((2 * _N, _D), jnp.float32),
        mesh=mesh,
        scratch_types=(
            [pltpu.VMEM((_CHUNK,), jnp.int32)] * 3
            + [pltpu.VMEM((_CHUNK,), jnp.int32)] * 3
            + [pltpu.VMEM((_CHUNK, _D), jnp.float32)] * 3
            + [pltpu.VMEM((_CHUNK, _D), jnp.float32)] * 3
            + [pltpu.VMEM_SHARED((_N, _D), jnp.float32)]
            + [pltpu.SemaphoreType.DMA] * 15
        ),
    )(_sc_edge_body)
    return f(x, e_l, src, dst)


def _node_body(x_ref, agg_ref, w_ref, b_ref, g_ref, bt_ref, o_ref):
    xa = x_ref[...] + agg_ref[0:_N, :] + agg_ref[_N:2 * _N, :]
    h = jnp.dot(xa, w_ref[...], preferred_element_type=jnp.float32) + b_ref[...]
    mu = jnp.mean(h, axis=0, keepdims=True)
    var = jnp.mean(h * h, axis=0, keepdims=True) - mu * mu
    hn = (h - mu) * lax.rsqrt(var + 1e-5) * g_ref[...] + bt_ref[...]
    o_ref[...] = jnp.where(hn >= 0, hn, 0.01 * hn)


def _node(x, agg2, w, b, g, bt):
    return pl.pallas_call(
        _node_body,
        out_shape=jax.ShapeDtypeStruct((_N, _D), jnp.float32),
    )(x, agg2, w, b.reshape(1, _D), g.reshape(1, _D), bt.reshape(1, _D))


def _final_body(h1_ref, h2_ref, h3_ref, brow_ref, bcol_ref, wl_ref, bl_ref,
                o_ref):
    w1 = wl_ref[0:_D, :]
    w2 = wl_ref[_D:2 * _D, :]
    w3 = wl_ref[2 * _D:3 * _D, :]
    w4 = wl_ref[3 * _D:4 * _D, :]
    h3 = h3_ref[...]
    t = (jnp.dot(h1_ref[...], w1, preferred_element_type=jnp.float32)
         + jnp.dot(h2_ref[...], w2, preferred_element_type=jnp.float32)
         + jnp.dot(h3, w3, preferred_element_type=jnp.float32))
    oh = (lax.broadcasted_iota(jnp.int32, (_G, _N), 0)
          == brow_ref[...]).astype(jnp.float32)
    pool = jnp.dot(oh, h3, preferred_element_type=jnp.float32)
    s = jnp.dot(pool, w4, preferred_element_type=jnp.float32)
    oht = (lax.broadcasted_iota(jnp.int32, (_N, _G), 1)
           == bcol_ref[...]).astype(jnp.float32)
    pooled = jnp.dot(oht, s, preferred_element_type=jnp.float32)
    z = t + pooled + bl_ref[...]
    o_ref[...] = 1.0 / (1.0 + jnp.exp(-z))


def _final(h1, h2, h3, batch, wl, bl):
    return pl.pallas_call(
        _final_body,
        out_shape=jax.ShapeDtypeStruct((_N, 1), jnp.float32),
    )(h1, h2, h3, batch.reshape(1, _N), batch.reshape(_N, 1), wl,
      bl.reshape(1, 1))


def kernel(x, edge_index, edge_attr, batch,
           W1e, b1e, W1, b1, g1, bt1,
           W2e, b2e, W2, b2, g2, bt2,
           W3e, b3e, W3, b3, g3, bt3,
           Wl, bl):
    src = edge_index[0]
    dst = edge_index[1]
    e1, e2, e3 = _edge_mlp(edge_attr, W1e, b1e, W2e, b2e, W3e, b3e)

    h = x
    hs = []
    for e_l, w, b, g, bt in ((e1, W1, b1, g1, bt1),
                             (e2, W2, b2, g2, bt2),
                             (e3, W3, b3, g3, bt3)):
        agg2 = _sc_edge(h, e_l, src, dst)
        h = _node(h, agg2, w, b, g, bt)
        hs.append(h)

    return _final(hs[0], hs[1], hs[2], batch, Wl, bl)
